# baseline (speedup 1.0000x reference)
import jax
import jax.numpy as jnp
from jax import lax
from jax.experimental import pallas as pl
from jax.experimental.pallas import tpu as pltpu
from jax.experimental.pallas import tpu_sc as plsc

NC = 2
NS = 16
NW = NC * NS
L = 16
CH = 128

F = 128
BN = 1024
NB = 2
NPASS = 2


def _qrsqrt(x):
    i = lax.bitcast_convert_type(x, jnp.int32)
    i = 0x5F3759DF - lax.shift_right_arithmetic(i, 1)
    y = lax.bitcast_convert_type(i, jnp.float32)
    for _ in range(3):
        y = y * (1.5 - 0.5 * x * y * y)
    return y


def _sc_mesh():
    return plsc.VectorSubcoreMesh(
        core_axis_name="c", subcore_axis_name="s", num_cores=NC, num_subcores=NS
    )


def _wvec(idxb, r, g):
    return plsc.bitcast(idxb[r, pl.ds(g * L, L)], jnp.float32)


def _make_deg(np_, tot_chunks):
    cpt = tot_chunks // NW
    seg = np_ // NS

    def body(idx_hbm, degp_hbm, idxb, deg_v, segb, accb, stage_sh):
        cid = lax.axis_index("c")
        sid = lax.axis_index("s")
        wid = sid * NC + cid

        pltpu.sync_copy(idx_hbm.at[pl.ds(wid * cpt * 3, cpt * 3)], idxb)

        def zero(i, c):
            deg_v[pl.ds(i * L, L)] = jnp.zeros((L,), jnp.float32)
            return c

        lax.fori_loop(0, np_ // L, zero, 0)

        def chunk(i, c):
            for g in range(CH // L):
                cv = idxb[3 * i + 1, pl.ds(g * L, L)]
                wv = _wvec(idxb, 3 * i + 2, g)
                plsc.addupdate_scatter(deg_v, [cv], wv)
            return c

        lax.fori_loop(0, cpt, chunk, 0)
        pltpu.sync_copy(deg_v, stage_sh.at[pl.ds(sid * np_, np_)])
        plsc.subcore_barrier()

        def zseg(i, c):
            accb[pl.ds(i * L, L)] = jnp.zeros((L,), jnp.float32)
            return c

        lax.fori_loop(0, seg // L, zseg, 0)
        for j in range(NS):
            pltpu.sync_copy(stage_sh.at[pl.ds(j * np_ + sid * seg, seg)], segb)

            def addseg(i, c):
                sl = pl.ds(i * L, L)
                accb[sl] = accb[sl] + segb[sl]
                return c

            lax.fori_loop(0, seg // L, addseg, 0)
        pltpu.sync_copy(accb, degp_hbm.at[pl.ds(cid * np_ + sid * seg, seg)])

    return pl.kernel(
        body,
        out_type=jax.ShapeDtypeStruct((NC * np_,), jnp.float32),
        mesh=_sc_mesh(),
        compiler_params=pltpu.CompilerParams(needs_layout_passes=False),
        scratch_types=[
            pltpu.VMEM((cpt * 3, CH), jnp.int32),
            pltpu.VMEM((np_,), jnp.float32),
            pltpu.VMEM((np_ // NS,), jnp.float32),
            pltpu.VMEM((np_ // NS,), jnp.float32),
            pltpu.VMEM_SHARED((NS * np_,), jnp.float32),
        ],
    )


def _make_dinv(np_, n_real):
    rows = np_ // NW

    def body(degp_hbm, dinv_hbm, degb0, degb1, dinvb):
        cid = lax.axis_index("c")
        sid = lax.axis_index("s")
        wid = sid * NC + cid
        base = wid * rows
        pltpu.sync_copy(degp_hbm.at[pl.ds(base, rows)], degb0)
        pltpu.sync_copy(degp_hbm.at[pl.ds(np_ + base, rows)], degb1)

        def grp(i, c):
            acc = degb0[pl.ds(i * L, L)] + degb1[pl.ds(i * L, L)]
            nvec = base + i * L + lax.iota(jnp.int32, 16)
            deg = acc + jnp.where(nvec < n_real, 1.0, 0.0)
            y = jnp.where(deg > 0.0, _qrsqrt(deg), 0.0)
            dinvb[pl.ds(i * L, L)] = y
            return c

        lax.fori_loop(0, rows // L, grp, 0)
        pltpu.sync_copy(dinvb, dinv_hbm.at[pl.ds(base, rows)])

    return pl.kernel(
        body,
        out_type=jax.ShapeDtypeStruct((np_,), jnp.float32),
        mesh=_sc_mesh(),
        compiler_params=pltpu.CompilerParams(needs_layout_passes=False),
        scratch_types=[
            pltpu.VMEM((rows,), jnp.float32),
            pltpu.VMEM((rows,), jnp.float32),
            pltpu.VMEM((rows,), jnp.float32),
        ],
    )


def _make_spmm(np_, tot_chunks):
    cpt = tot_chunks // NW
    cpp = cpt // NPASS
    assert cpp % NB == 0
    zrows = np_ // NS

    def body(p_hbm, idx_hbm, out_hbm, acc_sh, idxb,
             r0, r1, g0, g1, s0, s1):
        rows = (r0, r1)
        gsem = (g0, g1)
        ssem = (s0, s1)
        cid = lax.axis_index("c")
        sid = lax.axis_index("s")
        wid = sid * NC + cid

        def zero(j, c):
            for k in range(F // L):
                r0[j, pl.ds(k * L, L)] = jnp.zeros((L,), jnp.float32)
            return c

        lax.fori_loop(0, CH, zero, 0)
        for r in range(zrows // CH):
            pltpu.sync_copy(r0, acc_sh.at[pl.ds(sid * zrows + r * CH, CH)])
        plsc.subcore_barrier()

        for p in range(NPASS):
            pltpu.sync_copy(
                idx_hbm.at[pl.ds((wid * cpt + p * cpp) * 3, cpp * 3)], idxb
            )

            for b in range(NB - 1):
                pltpu.async_copy(p_hbm.at[idxb.at[3 * b]], rows[b], gsem[b])

            def step(t, c):
                for b in range(NB):
                    i = t * NB + b
                    bp = (b - 1) % NB
                    pltpu.make_async_copy(
                        p_hbm.at[idxb.at[3 * i]], rows[b], gsem[b]
                    ).wait()

                    def scale(g, c2, _b=b, _i=i):
                        wv = _wvec(idxb, 3 * _i + 2, g)
                        for j in range(L):
                            jj = g * L + j
                            wj = wv[j]
                            for k in range(F // L):
                                sl = pl.ds(k * L, L)
                                rows[_b][jj, sl] = rows[_b][jj, sl] * wj
                        return c2

                    lax.fori_loop(0, CH // L, scale, 0)
                    pltpu.async_copy(
                        rows[b], acc_sh.at[idxb.at[3 * i + 1]], ssem[b],
                        add=True,
                    )

                    def retire(_b=bp, _i=i):
                        pltpu.make_async_copy(
                            rows[_b], acc_sh.at[idxb.at[3 * (_i - 1) + 1]],
                            ssem[_b],
                        ).wait()

                    if b == 0:
                        pl.when(t > 0)(retire)
                    else:
                        retire()

                    jn = i + NB - 1

                    def refill(_b=bp, _j=jn):
                        pltpu.async_copy(
                            p_hbm.at[idxb.at[3 * _j]], rows[_b], gsem[_b]
                        )

                    pl.when(jn < cpp)(refill)
                return c

            lax.fori_loop(0, cpp // NB, step, 0)
            pltpu.make_async_copy(
                rows[NB - 1], acc_sh.at[idxb.at[3 * (cpp - 1) + 1]],
                ssem[NB - 1],
            ).wait()

        plsc.subcore_barrier()
        pltpu.sync_copy(
            acc_sh.at[pl.ds(sid * zrows, zrows)],
            out_hbm.at[cid, pl.ds(sid * zrows, zrows)],
        )

    return pl.kernel(
        body,
        out_type=jax.ShapeDtypeStruct((NC, np_, F), jnp.float32),
        mesh=_sc_mesh(),
        compiler_params=pltpu.CompilerParams(
            needs_layout_passes=False, use_tc_tiling_on_sc=False
        ),
        scratch_types=(
            [
                pltpu.VMEM_SHARED((np_, F), jnp.float32),
                pltpu.VMEM((cpt // NPASS * 3, CH), jnp.int32),
            ]
            + [pltpu.VMEM((CH, F), jnp.float32)] * NB
            + [pltpu.SemaphoreType.DMA] * (2 * NB)
        ),
    )


def _mm_body(x_ref, w_ref, dinv_ref, o_ref):
    o_ref[...] = (
        jnp.dot(x_ref[...], w_ref[...], preferred_element_type=jnp.float32)
        * dinv_ref[...]
    )


def _mid_body(s_ref, p_ref, dinv_ref, b1_ref, w2_ref, o_ref):
    z = (s_ref[0] + s_ref[1] + p_ref[...]) * dinv_ref[...] + b1_ref[...]
    h = jnp.maximum(z, 0.0)
    o_ref[...] = (
        jnp.dot(h, w2_ref[...], preferred_element_type=jnp.float32)
        * dinv_ref[...]
    )


def _out_body(s_ref, p_ref, dinv_ref, b2_ref, o_ref):
    o_ref[...] = (
        (s_ref[0] + s_ref[1] + p_ref[...]) * dinv_ref[...] + b2_ref[...]
    )


def _tc_matmul(xp, w, dinv2d, np_):
    return pl.pallas_call(
        _mm_body,
        grid=(np_ // BN,),
        in_specs=[
            pl.BlockSpec((BN, F), lambda i: (i, 0)),
            pl.BlockSpec((F, F), lambda i: (0, 0)),
            pl.BlockSpec((BN, 1), lambda i: (i, 0)),
        ],
        out_specs=pl.BlockSpec((BN, F), lambda i: (i, 0)),
        out_shape=jax.ShapeDtypeStruct((np_, F), jnp.float32),
    )(xp, w, dinv2d)


def _tc_mid(s, p, dinv2d, b1, w2, np_):
    return pl.pallas_call(
        _mid_body,
        grid=(np_ // BN,),
        in_specs=[
            pl.BlockSpec((NC, BN, F), lambda i: (0, i, 0)),
            pl.BlockSpec((BN, F), lambda i: (i, 0)),
            pl.BlockSpec((BN, 1), lambda i: (i, 0)),
            pl.BlockSpec((1, F), lambda i: (0, 0)),
            pl.BlockSpec((F, F), lambda i: (0, 0)),
        ],
        out_specs=pl.BlockSpec((BN, F), lambda i: (i, 0)),
        out_shape=jax.ShapeDtypeStruct((np_, F), jnp.float32),
    )(s, p, dinv2d, b1, w2)


def _tc_out(s, p, dinv2d, b2, np_):
    return pl.pallas_call(
        _out_body,
        grid=(np_ // BN,),
        in_specs=[
            pl.BlockSpec((NC, BN, F), lambda i: (0, i, 0)),
            pl.BlockSpec((BN, F), lambda i: (i, 0)),
            pl.BlockSpec((BN, 1), lambda i: (i, 0)),
            pl.BlockSpec((1, F), lambda i: (0, 0)),
        ],
        out_specs=pl.BlockSpec((BN, F), lambda i: (i, 0)),
        out_shape=jax.ShapeDtypeStruct((np_, F), jnp.float32),
    )(s, p, dinv2d, b2)


def kernel(x, attn_edge_index, attn_edge_weight, W1, b1, W2, b2):
    n, f = x.shape
    e = attn_edge_weight.shape[0]
    assert f == F

    np_ = ((n + NW * L - 1) // (NW * L)) * (NW * L)
    align = NW * NPASS * NB * CH
    e_pad = ((e + align - 1) // align) * align
    tot_chunks = e_pad // CH

    row = attn_edge_index[0]
    col = attn_edge_index[1]
    rowp = jnp.pad(row, (0, e_pad - e)).reshape(tot_chunks, CH)
    colp = jnp.pad(col, (0, e_pad - e)).reshape(tot_chunks, CH)
    wbits = lax.bitcast_convert_type(
        jnp.pad(attn_edge_weight, (0, e_pad - e)), jnp.int32
    ).reshape(tot_chunks, CH)
    idx3 = jnp.stack([rowp, colp, wbits], axis=1).reshape(tot_chunks * 3, CH)
    xp = jnp.pad(x, ((0, np_ - n), (0, 0)))

    degp = _make_deg(np_, tot_chunks)(idx3)
    dinv = _make_dinv(np_, n)(degp)
    dinv2d = dinv.reshape(np_, 1)
    p1 = _tc_matmul(xp, W1, dinv2d, np_)
    spmm = _make_spmm(np_, tot_chunks)
    s1 = spmm(p1, idx3)
    p2 = _tc_mid(s1, p1, dinv2d, b1.reshape(1, F), W2, np_)
    s2 = spmm(p2, idx3)
    outp = _tc_out(s2, p2, dinv2d, b2.reshape(1, F), np_)
    return outp[:n]


# speedup vs baseline: 1.5872x; 1.5872x over previous
import jax
import jax.numpy as jnp
from jax import lax
from jax.experimental import pallas as pl
from jax.experimental.pallas import tpu as pltpu
from jax.experimental.pallas import tpu_sc as plsc

NC = 2
NS = 16
NW = NC * NS
L = 16
CH = 128

F = 128
FH = F // 2
BN = 1024
NB = 4
NPASS = 2


def _qrsqrt(x):
    i = lax.bitcast_convert_type(x, jnp.int32)
    i = 0x5F3759DF - lax.shift_right_arithmetic(i, 1)
    y = lax.bitcast_convert_type(i, jnp.float32)
    for _ in range(3):
        y = y * (1.5 - 0.5 * x * y * y)
    return y


def _sc_mesh():
    return plsc.VectorSubcoreMesh(
        core_axis_name="c", subcore_axis_name="s", num_cores=NC, num_subcores=NS
    )


def _wvec(idxb, r, g):
    return plsc.bitcast(idxb[r, pl.ds(g * L, L)], jnp.float32)


def _make_deg(np_, tot_chunks):
    cpt = tot_chunks // NW
    seg = np_ // NS

    def body(idx_hbm, degp_hbm, idxb, deg_v, segb, accb, stage_sh):
        cid = lax.axis_index("c")
        sid = lax.axis_index("s")
        wid = sid * NC + cid

        pltpu.sync_copy(idx_hbm.at[pl.ds(wid * cpt * 3, cpt * 3)], idxb)

        def zero(i, c):
            deg_v[pl.ds(i * L, L)] = jnp.zeros((L,), jnp.float32)
            return c

        lax.fori_loop(0, np_ // L, zero, 0)

        def chunk(i, c):
            for g in range(CH // L):
                cv = idxb[3 * i + 1, pl.ds(g * L, L)]
                wv = _wvec(idxb, 3 * i + 2, g)
                plsc.addupdate_scatter(deg_v, [cv], wv)
            return c

        lax.fori_loop(0, cpt, chunk, 0)
        pltpu.sync_copy(deg_v, stage_sh.at[pl.ds(sid * np_, np_)])
        plsc.subcore_barrier()

        def zseg(i, c):
            accb[pl.ds(i * L, L)] = jnp.zeros((L,), jnp.float32)
            return c

        lax.fori_loop(0, seg // L, zseg, 0)
        for j in range(NS):
            pltpu.sync_copy(stage_sh.at[pl.ds(j * np_ + sid * seg, seg)], segb)

            def addseg(i, c):
                sl = pl.ds(i * L, L)
                accb[sl] = accb[sl] + segb[sl]
                return c

            lax.fori_loop(0, seg // L, addseg, 0)
        pltpu.sync_copy(accb, degp_hbm.at[pl.ds(cid * np_ + sid * seg, seg)])

    return pl.kernel(
        body,
        out_type=jax.ShapeDtypeStruct((NC * np_,), jnp.float32),
        mesh=_sc_mesh(),
        compiler_params=pltpu.CompilerParams(needs_layout_passes=False),
        scratch_types=[
            pltpu.VMEM((cpt * 3, CH), jnp.int32),
            pltpu.VMEM((np_,), jnp.float32),
            pltpu.VMEM((np_ // NS,), jnp.float32),
            pltpu.VMEM((np_ // NS,), jnp.float32),
            pltpu.VMEM_SHARED((NS * np_,), jnp.float32),
        ],
    )


def _make_dinv(np_, n_real):
    rows = np_ // NW

    def body(degp_hbm, dinv_hbm, degb0, degb1, dinvb):
        cid = lax.axis_index("c")
        sid = lax.axis_index("s")
        wid = sid * NC + cid
        base = wid * rows
        pltpu.sync_copy(degp_hbm.at[pl.ds(base, rows)], degb0)
        pltpu.sync_copy(degp_hbm.at[pl.ds(np_ + base, rows)], degb1)

        def grp(i, c):
            acc = degb0[pl.ds(i * L, L)] + degb1[pl.ds(i * L, L)]
            nvec = base + i * L + lax.iota(jnp.int32, 16)
            deg = acc + jnp.where(nvec < n_real, 1.0, 0.0)
            y = jnp.where(deg > 0.0, _qrsqrt(deg), 0.0)
            dinvb[pl.ds(i * L, L)] = y
            return c

        lax.fori_loop(0, rows // L, grp, 0)
        pltpu.sync_copy(dinvb, dinv_hbm.at[pl.ds(base, rows)])

    return pl.kernel(
        body,
        out_type=jax.ShapeDtypeStruct((np_,), jnp.float32),
        mesh=_sc_mesh(),
        compiler_params=pltpu.CompilerParams(needs_layout_passes=False),
        scratch_types=[
            pltpu.VMEM((rows,), jnp.float32),
            pltpu.VMEM((rows,), jnp.float32),
            pltpu.VMEM((rows,), jnp.float32),
        ],
    )


def _make_spmm(np_, tot_chunks):
    cpt = tot_chunks // NS
    cpp = cpt // NPASS
    assert cpp % NB == 0
    zrows = np_ // NS

    def body(p_hbm, idx_hbm, out_hbm, acc_sh, idxb,
             r0, r1, r2, r3, h0, h1, h2, h3,
             g0, g1, g2, g3, s0, s1, s2, s3):
        rows = (r0, r1, r2, r3)
        hbuf = (h0, h1, h2, h3)
        gsem = (g0, g1, g2, g3)
        ssem = (s0, s1, s2, s3)
        cid = lax.axis_index("c")
        sid = lax.axis_index("s")

        def zero(j, c):
            for k in range(FH // L):
                r0[j, pl.ds(k * L, L)] = jnp.zeros((L,), jnp.float32)
            return c

        lax.fori_loop(0, CH, zero, 0)
        for r in range(zrows // CH):
            pltpu.sync_copy(r0, acc_sh.at[pl.ds(sid * zrows + r * CH, CH)])
        plsc.subcore_barrier()

        for p in range(NPASS):
            pltpu.sync_copy(
                idx_hbm.at[pl.ds((sid * cpt + p * cpp) * 3, cpp * 3)], idxb
            )

            def bias(i, c):
                off = cid * np_
                for g in range(CH // L):
                    sl = pl.ds(g * L, L)
                    idxb[3 * i, sl] = idxb[3 * i, sl] + off
                return c

            lax.fori_loop(0, cpp, bias, 0)

            for b in range(NB - 1):
                pltpu.async_copy(p_hbm.at[idxb.at[3 * b]], hbuf[b], gsem[b])

            def step(t, c):
                for b in range(NB):
                    i = t * NB + b
                    bp = (b - 1) % NB
                    pltpu.make_async_copy(
                        p_hbm.at[idxb.at[3 * i]], hbuf[b], gsem[b]
                    ).wait()

                    def scale(g, c2, _b=b, _i=i):
                        wv = _wvec(idxb, 3 * _i + 2, g)
                        msk = jnp.full((L,), -65536, jnp.int32)
                        for j in range(L):
                            jj = g * L + j
                            wj = wv[j]
                            for k in range(FH // (2 * L)):
                                v = hbuf[_b][jj, pl.ds(k * L, L)]
                                ev = plsc.bitcast(
                                    lax.shift_left(v, 16), jnp.float32
                                )
                                od = plsc.bitcast(v & msk, jnp.float32)
                                fb = 2 * k * L
                                rows[_b][jj, pl.ds(fb, L)] = ev * wj
                                rows[_b][jj, pl.ds(fb + L, L)] = od * wj
                        return c2

                    lax.fori_loop(0, CH // L, scale, 0)
                    pltpu.async_copy(
                        rows[b], acc_sh.at[idxb.at[3 * i + 1]], ssem[b],
                        add=True,
                    )

                    def retire(_b=bp, _i=i):
                        pltpu.make_async_copy(
                            rows[_b], acc_sh.at[idxb.at[3 * (_i - 1) + 1]],
                            ssem[_b],
                        ).wait()

                    if b == 0:
                        pl.when(t > 0)(retire)
                    else:
                        retire()

                    jn = i + NB - 1

                    def refill(_b=bp, _j=jn):
                        pltpu.async_copy(
                            p_hbm.at[idxb.at[3 * _j]], hbuf[_b], gsem[_b]
                        )

                    pl.when(jn < cpp)(refill)
                return c

            lax.fori_loop(0, cpp // NB, step, 0)
            pltpu.make_async_copy(
                rows[NB - 1], acc_sh.at[idxb.at[3 * (cpp - 1) + 1]],
                ssem[NB - 1],
            ).wait()

        plsc.subcore_barrier()
        pltpu.sync_copy(
            acc_sh.at[pl.ds(sid * zrows, zrows)],
            out_hbm.at[cid, pl.ds(sid * zrows, zrows)],
        )

    return pl.kernel(
        body,
        out_type=jax.ShapeDtypeStruct((NC, np_, FH), jnp.float32),
        mesh=_sc_mesh(),
        compiler_params=pltpu.CompilerParams(
            needs_layout_passes=False, use_tc_tiling_on_sc=False
        ),
        scratch_types=(
            [
                pltpu.VMEM_SHARED((np_, FH), jnp.float32),
                pltpu.VMEM((cpt // NPASS * 3, CH), jnp.int32),
            ]
            + [pltpu.VMEM((CH, FH), jnp.float32)] * NB
            + [pltpu.VMEM((CH, FH // 2), jnp.int32)] * NB
            + [pltpu.SemaphoreType.DMA] * (2 * NB)
        ),
    )


def _mm_body(x_ref, w_ref, dinv_ref, o_ref, oh_ref):
    r = (
        jnp.dot(x_ref[...], w_ref[...], preferred_element_type=jnp.float32)
        * dinv_ref[...]
    )
    o_ref[0] = r[:, :FH]
    o_ref[1] = r[:, FH:]
    oh_ref[0] = r[:, :FH].astype(jnp.bfloat16)
    oh_ref[1] = r[:, FH:].astype(jnp.bfloat16)


def _mid_body(s_ref, p_ref, dinv_ref, b1_ref, w2_ref, o_ref, oh_ref):
    z = jnp.concatenate(
        [s_ref[0] + p_ref[0], s_ref[1] + p_ref[1]], axis=-1
    ) * dinv_ref[...] + b1_ref[...]
    h = jnp.maximum(z, 0.0)
    r = (
        jnp.dot(h, w2_ref[...], preferred_element_type=jnp.float32)
        * dinv_ref[...]
    )
    o_ref[0] = r[:, :FH]
    o_ref[1] = r[:, FH:]
    oh_ref[0] = r[:, :FH].astype(jnp.bfloat16)
    oh_ref[1] = r[:, FH:].astype(jnp.bfloat16)


def _out_body(s_ref, p_ref, dinv_ref, b2_ref, o_ref):
    o_ref[...] = jnp.concatenate(
        [s_ref[0] + p_ref[0], s_ref[1] + p_ref[1]], axis=-1
    ) * dinv_ref[...] + b2_ref[...]


def _tc_matmul(xp, w, dinv2d, np_):
    return pl.pallas_call(
        _mm_body,
        grid=(np_ // BN,),
        in_specs=[
            pl.BlockSpec((BN, F), lambda i: (i, 0)),
            pl.BlockSpec((F, F), lambda i: (0, 0)),
            pl.BlockSpec((BN, 1), lambda i: (i, 0)),
        ],
        out_specs=[
            pl.BlockSpec((NC, BN, FH), lambda i: (0, i, 0)),
            pl.BlockSpec((NC, BN, FH), lambda i: (0, i, 0)),
        ],
        out_shape=[
            jax.ShapeDtypeStruct((NC, np_, FH), jnp.float32),
            jax.ShapeDtypeStruct((NC, np_, FH), jnp.bfloat16),
        ],
    )(xp, w, dinv2d)


def _tc_mid(s, p, dinv2d, b1, w2, np_):
    return pl.pallas_call(
        _mid_body,
        grid=(np_ // BN,),
        in_specs=[
            pl.BlockSpec((NC, BN, FH), lambda i: (0, i, 0)),
            pl.BlockSpec((NC, BN, FH), lambda i: (0, i, 0)),
            pl.BlockSpec((BN, 1), lambda i: (i, 0)),
            pl.BlockSpec((1, F), lambda i: (0, 0)),
            pl.BlockSpec((F, F), lambda i: (0, 0)),
        ],
        out_specs=[
            pl.BlockSpec((NC, BN, FH), lambda i: (0, i, 0)),
            pl.BlockSpec((NC, BN, FH), lambda i: (0, i, 0)),
        ],
        out_shape=[
            jax.ShapeDtypeStruct((NC, np_, FH), jnp.float32),
            jax.ShapeDtypeStruct((NC, np_, FH), jnp.bfloat16),
        ],
    )(s, p, dinv2d, b1, w2)


def _tc_out(s, p, dinv2d, b2, np_):
    return pl.pallas_call(
        _out_body,
        grid=(np_ // BN,),
        in_specs=[
            pl.BlockSpec((NC, BN, FH), lambda i: (0, i, 0)),
            pl.BlockSpec((NC, BN, FH), lambda i: (0, i, 0)),
            pl.BlockSpec((BN, 1), lambda i: (i, 0)),
            pl.BlockSpec((1, F), lambda i: (0, 0)),
        ],
        out_specs=pl.BlockSpec((BN, F), lambda i: (i, 0)),
        out_shape=jax.ShapeDtypeStruct((np_, F), jnp.float32),
    )(s, p, dinv2d, b2)


def kernel(x, attn_edge_index, attn_edge_weight, W1, b1, W2, b2):
    n, f = x.shape
    e = attn_edge_weight.shape[0]
    assert f == F

    np_ = ((n + NW * L - 1) // (NW * L)) * (NW * L)
    align = NS * NPASS * NB * CH
    e_pad = ((e + align - 1) // align) * align
    tot_chunks = e_pad // CH

    row = attn_edge_index[0]
    col = attn_edge_index[1]
    rowp = jnp.pad(row, (0, e_pad - e)).reshape(tot_chunks, CH)
    colp = jnp.pad(col, (0, e_pad - e)).reshape(tot_chunks, CH)
    wbits = lax.bitcast_convert_type(
        jnp.pad(attn_edge_weight, (0, e_pad - e)), jnp.int32
    ).reshape(tot_chunks, CH)
    idx3 = jnp.stack([rowp, colp, wbits], axis=1).reshape(tot_chunks * 3, CH)
    xp = jnp.pad(x, ((0, np_ - n), (0, 0)))

    def pair_table(ph):
        t = ph.reshape(NC * np_, 2, 2, L).transpose(0, 1, 3, 2)
        return lax.bitcast_convert_type(t, jnp.int32).reshape(
            NC * np_, FH // 2
        )

    degp = _make_deg(np_, tot_chunks)(idx3)
    dinv = _make_dinv(np_, n)(degp)
    dinv2d = dinv.reshape(np_, 1)
    p1, p1h = _tc_matmul(xp, W1, dinv2d, np_)
    spmm = _make_spmm(np_, tot_chunks)
    s1 = spmm(pair_table(p1h), idx3)
    p2, p2h = _tc_mid(s1, p1, dinv2d, b1.reshape(1, F), W2, np_)
    s2 = spmm(pair_table(p2h), idx3)
    outp = _tc_out(s2, p2, dinv2d, b2.reshape(1, F), np_)
    return outp[:n]


# speedup vs baseline: 2.0245x; 1.2755x over previous
import jax
import jax.numpy as jnp
from jax import lax
from jax.experimental import pallas as pl
from jax.experimental.pallas import tpu as pltpu
from jax.experimental.pallas import tpu_sc as plsc

NC = 2
NS = 16
NW = NC * NS
L = 16
CH = 128

F = 128
FH = F // 2
BN = 1024
NB = 4
NPASS = 2


def _qrsqrt(x):
    i = lax.bitcast_convert_type(x, jnp.int32)
    i = 0x5F3759DF - lax.shift_right_arithmetic(i, 1)
    y = lax.bitcast_convert_type(i, jnp.float32)
    for _ in range(3):
        y = y * (1.5 - 0.5 * x * y * y)
    return y


def _sc_mesh():
    return plsc.VectorSubcoreMesh(
        core_axis_name="c", subcore_axis_name="s", num_cores=NC, num_subcores=NS
    )


def _wvec(idxb, r, g):
    return plsc.bitcast(idxb[r, pl.ds(g * L, L)], jnp.float32)


def _make_deg(np_, tot_chunks):
    cpt = tot_chunks // NW
    seg = np_ // NS

    def body(idx_hbm, degp_hbm, idxb, deg_v, segb, accb, stage_sh):
        cid = lax.axis_index("c")
        sid = lax.axis_index("s")
        wid = sid * NC + cid

        pltpu.sync_copy(idx_hbm.at[pl.ds(wid * cpt * 3, cpt * 3)], idxb)

        def zero(i, c):
            deg_v[pl.ds(i * L, L)] = jnp.zeros((L,), jnp.float32)
            return c

        lax.fori_loop(0, np_ // L, zero, 0)

        def chunk(i, c):
            for g in range(CH // L):
                cv = idxb[3 * i + 1, pl.ds(g * L, L)]
                wv = _wvec(idxb, 3 * i + 2, g)
                plsc.addupdate_scatter(deg_v, [cv], wv)
            return c

        lax.fori_loop(0, cpt, chunk, 0)
        pltpu.sync_copy(deg_v, stage_sh.at[pl.ds(sid * np_, np_)])
        plsc.subcore_barrier()

        def zseg(i, c):
            accb[pl.ds(i * L, L)] = jnp.zeros((L,), jnp.float32)
            return c

        lax.fori_loop(0, seg // L, zseg, 0)
        for j in range(NS):
            pltpu.sync_copy(stage_sh.at[pl.ds(j * np_ + sid * seg, seg)], segb)

            def addseg(i, c):
                sl = pl.ds(i * L, L)
                accb[sl] = accb[sl] + segb[sl]
                return c

            lax.fori_loop(0, seg // L, addseg, 0)
        pltpu.sync_copy(accb, degp_hbm.at[pl.ds(cid * np_ + sid * seg, seg)])

    return pl.kernel(
        body,
        out_type=jax.ShapeDtypeStruct((NC * np_,), jnp.float32),
        mesh=_sc_mesh(),
        compiler_params=pltpu.CompilerParams(needs_layout_passes=False),
        scratch_types=[
            pltpu.VMEM((cpt * 3, CH), jnp.int32),
            pltpu.VMEM((np_,), jnp.float32),
            pltpu.VMEM((np_ // NS,), jnp.float32),
            pltpu.VMEM((np_ // NS,), jnp.float32),
            pltpu.VMEM_SHARED((NS * np_,), jnp.float32),
        ],
    )


def _make_dinv(np_, n_real):
    rows = np_ // NW

    def body(degp_hbm, dinv_hbm, degb0, degb1, dinvb):
        cid = lax.axis_index("c")
        sid = lax.axis_index("s")
        wid = sid * NC + cid
        base = wid * rows
        pltpu.sync_copy(degp_hbm.at[pl.ds(base, rows)], degb0)
        pltpu.sync_copy(degp_hbm.at[pl.ds(np_ + base, rows)], degb1)

        def grp(i, c):
            acc = degb0[pl.ds(i * L, L)] + degb1[pl.ds(i * L, L)]
            nvec = base + i * L + lax.iota(jnp.int32, 16)
            deg = acc + jnp.where(nvec < n_real, 1.0, 0.0)
            y = jnp.where(deg > 0.0, _qrsqrt(deg), 0.0)
            dinvb[pl.ds(i * L, L)] = y
            return c

        lax.fori_loop(0, rows // L, grp, 0)
        pltpu.sync_copy(dinvb, dinv_hbm.at[pl.ds(base, rows)])

    return pl.kernel(
        body,
        out_type=jax.ShapeDtypeStruct((np_,), jnp.float32),
        mesh=_sc_mesh(),
        compiler_params=pltpu.CompilerParams(needs_layout_passes=False),
        scratch_types=[
            pltpu.VMEM((rows,), jnp.float32),
            pltpu.VMEM((rows,), jnp.float32),
            pltpu.VMEM((rows,), jnp.float32),
        ],
    )


def _make_spmm(np_, tot_chunks):
    cpt = tot_chunks // NS
    cpp = cpt // NPASS
    assert cpp % NB == 0
    zrows = np_ // NS

    def body(p_hbm, idx_hbm, out_hbm, acc0_sh, acc1_sh, idxb,
             ra0, ra1, ra2, ra3, rb0, rb1, rb2, rb3, h0, h1, h2, h3,
             g0, g1, g2, g3, sa0, sa1, sa2, sa3, sb0, sb1, sb2, sb3):
        rowsa = (ra0, ra1, ra2, ra3)
        rowsb = (rb0, rb1, rb2, rb3)
        hbuf = (h0, h1, h2, h3)
        gsem = (g0, g1, g2, g3)
        ssema = (sa0, sa1, sa2, sa3)
        ssemb = (sb0, sb1, sb2, sb3)
        cid = lax.axis_index("c")
        sid = lax.axis_index("s")

        def zero(j, c):
            for k in range(FH // 2 // L):
                ra0[j, pl.ds(k * L, L)] = jnp.zeros((L,), jnp.float32)
            return c

        lax.fori_loop(0, CH, zero, 0)
        for r in range(zrows // CH):
            pltpu.sync_copy(ra0, acc0_sh.at[pl.ds(sid * zrows + r * CH, CH)])
            pltpu.sync_copy(ra0, acc1_sh.at[pl.ds(sid * zrows + r * CH, CH)])
        plsc.subcore_barrier()

        for p in range(NPASS):
            pltpu.sync_copy(
                idx_hbm.at[pl.ds((sid * cpt + p * cpp) * 3, cpp * 3)], idxb
            )

            def bias(i, c):
                off = cid * np_
                for g in range(CH // L):
                    sl = pl.ds(g * L, L)
                    idxb[3 * i, sl] = idxb[3 * i, sl] + off
                return c

            lax.fori_loop(0, cpp, bias, 0)

            for b in range(NB - 1):
                pltpu.async_copy(p_hbm.at[idxb.at[3 * b]], hbuf[b], gsem[b])

            def step(t, c):
                for b in range(NB):
                    i = t * NB + b
                    bp = (b - 1) % NB
                    pltpu.make_async_copy(
                        p_hbm.at[idxb.at[3 * i]], hbuf[b], gsem[b]
                    ).wait()

                    def scale(g, c2, _b=b, _i=i):
                        wv = _wvec(idxb, 3 * _i + 2, g)
                        msk = jnp.full((L,), -65536, jnp.int32)
                        for j in range(L):
                            jj = g * L + j
                            wj = wv[j]
                            for k in range(FH // (2 * L)):
                                v = hbuf[_b][jj, pl.ds(k * L, L)]
                                ev = plsc.bitcast(
                                    lax.shift_left(v, 16), jnp.float32
                                )
                                od = plsc.bitcast(v & msk, jnp.float32)
                                dst = rowsa[_b] if k == 0 else rowsb[_b]
                                dst[jj, pl.ds(0, L)] = ev * wj
                                dst[jj, pl.ds(L, L)] = od * wj
                        return c2

                    lax.fori_loop(0, CH // L, scale, 0)
                    pltpu.async_copy(
                        rowsa[b], acc0_sh.at[idxb.at[3 * i + 1]], ssema[b],
                        add=True,
                    )
                    pltpu.async_copy(
                        rowsb[b], acc1_sh.at[idxb.at[3 * i + 1]], ssemb[b],
                        add=True,
                    )

                    def retire(_b=bp, _i=i):
                        pltpu.make_async_copy(
                            rowsa[_b], acc0_sh.at[idxb.at[3 * (_i - 1) + 1]],
                            ssema[_b],
                        ).wait()
                        pltpu.make_async_copy(
                            rowsb[_b], acc1_sh.at[idxb.at[3 * (_i - 1) + 1]],
                            ssemb[_b],
                        ).wait()

                    if b == 0:
                        pl.when(t > 0)(retire)
                    else:
                        retire()

                    jn = i + NB - 1

                    def refill(_b=bp, _j=jn):
                        pltpu.async_copy(
                            p_hbm.at[idxb.at[3 * _j]], hbuf[_b], gsem[_b]
                        )

                    pl.when(jn < cpp)(refill)
                return c

            lax.fori_loop(0, cpp // NB, step, 0)
            pltpu.make_async_copy(
                rowsa[NB - 1], acc0_sh.at[idxb.at[3 * (cpp - 1) + 1]],
                ssema[NB - 1],
            ).wait()
            pltpu.make_async_copy(
                rowsb[NB - 1], acc1_sh.at[idxb.at[3 * (cpp - 1) + 1]],
                ssemb[NB - 1],
            ).wait()

        plsc.subcore_barrier()
        pltpu.sync_copy(
            acc0_sh.at[pl.ds(sid * zrows, zrows)],
            out_hbm.at[cid, 0, pl.ds(sid * zrows, zrows)],
        )
        pltpu.sync_copy(
            acc1_sh.at[pl.ds(sid * zrows, zrows)],
            out_hbm.at[cid, 1, pl.ds(sid * zrows, zrows)],
        )

    return pl.kernel(
        body,
        out_type=jax.ShapeDtypeStruct((NC, 2, np_, FH // 2), jnp.float32),
        mesh=_sc_mesh(),
        compiler_params=pltpu.CompilerParams(
            needs_layout_passes=False, use_tc_tiling_on_sc=False
        ),
        scratch_types=(
            [
                pltpu.VMEM_SHARED((np_, FH // 2), jnp.float32),
                pltpu.VMEM_SHARED((np_, FH // 2), jnp.float32),
                pltpu.VMEM((cpt // NPASS * 3, CH), jnp.int32),
            ]
            + [pltpu.VMEM((CH, FH // 2), jnp.float32)] * (2 * NB)
            + [pltpu.VMEM((CH, FH // 2), jnp.int32)] * NB
            + [pltpu.SemaphoreType.DMA] * (3 * NB)
        ),
    )


def _mm_body(x_ref, w_ref, dinv_ref, o_ref, oh_ref):
    r = (
        jnp.dot(x_ref[...], w_ref[...], preferred_element_type=jnp.float32)
        * dinv_ref[...]
    )
    o_ref[0] = r[:, :FH]
    o_ref[1] = r[:, FH:]
    oh_ref[0] = r[:, :FH].astype(jnp.bfloat16)
    oh_ref[1] = r[:, FH:].astype(jnp.bfloat16)


def _mid_body(s_ref, p_ref, dinv_ref, b1_ref, w2_ref, o_ref, oh_ref):
    s_full = jnp.concatenate(
        [s_ref[0, 0], s_ref[0, 1], s_ref[1, 0], s_ref[1, 1]], axis=-1
    )
    z = (
        s_full + jnp.concatenate([p_ref[0], p_ref[1]], axis=-1)
    ) * dinv_ref[...] + b1_ref[...]
    h = jnp.maximum(z, 0.0)
    r = (
        jnp.dot(h, w2_ref[...], preferred_element_type=jnp.float32)
        * dinv_ref[...]
    )
    o_ref[0] = r[:, :FH]
    o_ref[1] = r[:, FH:]
    oh_ref[0] = r[:, :FH].astype(jnp.bfloat16)
    oh_ref[1] = r[:, FH:].astype(jnp.bfloat16)


def _out_body(s_ref, p_ref, dinv_ref, b2_ref, o_ref):
    s_full = jnp.concatenate(
        [s_ref[0, 0], s_ref[0, 1], s_ref[1, 0], s_ref[1, 1]], axis=-1
    )
    o_ref[...] = (
        s_full + jnp.concatenate([p_ref[0], p_ref[1]], axis=-1)
    ) * dinv_ref[...] + b2_ref[...]


def _tc_matmul(xp, w, dinv2d, np_):
    return pl.pallas_call(
        _mm_body,
        grid=(np_ // BN,),
        in_specs=[
            pl.BlockSpec((BN, F), lambda i: (i, 0)),
            pl.BlockSpec((F, F), lambda i: (0, 0)),
            pl.BlockSpec((BN, 1), lambda i: (i, 0)),
        ],
        out_specs=[
            pl.BlockSpec((NC, BN, FH), lambda i: (0, i, 0)),
            pl.BlockSpec((NC, BN, FH), lambda i: (0, i, 0)),
        ],
        out_shape=[
            jax.ShapeDtypeStruct((NC, np_, FH), jnp.float32),
            jax.ShapeDtypeStruct((NC, np_, FH), jnp.bfloat16),
        ],
    )(xp, w, dinv2d)


def _tc_mid(s, p, dinv2d, b1, w2, np_):
    return pl.pallas_call(
        _mid_body,
        grid=(np_ // BN,),
        in_specs=[
            pl.BlockSpec((NC, 2, BN, FH // 2), lambda i: (0, 0, i, 0)),
            pl.BlockSpec((NC, BN, FH), lambda i: (0, i, 0)),
            pl.BlockSpec((BN, 1), lambda i: (i, 0)),
            pl.BlockSpec((1, F), lambda i: (0, 0)),
            pl.BlockSpec((F, F), lambda i: (0, 0)),
        ],
        out_specs=[
            pl.BlockSpec((NC, BN, FH), lambda i: (0, i, 0)),
            pl.BlockSpec((NC, BN, FH), lambda i: (0, i, 0)),
        ],
        out_shape=[
            jax.ShapeDtypeStruct((NC, np_, FH), jnp.float32),
            jax.ShapeDtypeStruct((NC, np_, FH), jnp.bfloat16),
        ],
    )(s, p, dinv2d, b1, w2)


def _tc_out(s, p, dinv2d, b2, np_):
    return pl.pallas_call(
        _out_body,
        grid=(np_ // BN,),
        in_specs=[
            pl.BlockSpec((NC, 2, BN, FH // 2), lambda i: (0, 0, i, 0)),
            pl.BlockSpec((NC, BN, FH), lambda i: (0, i, 0)),
            pl.BlockSpec((BN, 1), lambda i: (i, 0)),
            pl.BlockSpec((1, F), lambda i: (0, 0)),
        ],
        out_specs=pl.BlockSpec((BN, F), lambda i: (i, 0)),
        out_shape=jax.ShapeDtypeStruct((np_, F), jnp.float32),
    )(s, p, dinv2d, b2)


def kernel(x, attn_edge_index, attn_edge_weight, W1, b1, W2, b2):
    n, f = x.shape
    e = attn_edge_weight.shape[0]
    assert f == F

    np_ = ((n + NW * L - 1) // (NW * L)) * (NW * L)
    align = NS * NPASS * NB * CH
    e_pad = ((e + align - 1) // align) * align
    tot_chunks = e_pad // CH

    row = attn_edge_index[0]
    col = attn_edge_index[1]
    rowp = jnp.pad(row, (0, e_pad - e)).reshape(tot_chunks, CH)
    colp = jnp.pad(col, (0, e_pad - e)).reshape(tot_chunks, CH)
    wbits = lax.bitcast_convert_type(
        jnp.pad(attn_edge_weight, (0, e_pad - e)), jnp.int32
    ).reshape(tot_chunks, CH)
    idx3 = jnp.stack([rowp, colp, wbits], axis=1).reshape(tot_chunks * 3, CH)
    xp = jnp.pad(x, ((0, np_ - n), (0, 0)))

    def pair_table(ph):
        t = ph.reshape(NC * np_, 2, 2, L).transpose(0, 1, 3, 2)
        return lax.bitcast_convert_type(t, jnp.int32).reshape(
            NC * np_, FH // 2
        )

    degp = _make_deg(np_, tot_chunks)(idx3)
    dinv = _make_dinv(np_, n)(degp)
    dinv2d = dinv.reshape(np_, 1)
    p1, p1h = _tc_matmul(xp, W1, dinv2d, np_)
    spmm = _make_spmm(np_, tot_chunks)
    s1 = spmm(pair_table(p1h), idx3)
    p2, p2h = _tc_mid(s1, p1, dinv2d, b1.reshape(1, F), W2, np_)
    s2 = spmm(pair_table(p2h), idx3)
    outp = _tc_out(s2, p2, dinv2d, b2.reshape(1, F), np_)
    return outp[:n]
